# baseline (speedup 1.0000x reference)
import functools

import jax
import jax.numpy as jnp
from jax import lax
from jax.experimental import pallas as pl
from jax.experimental.pallas import tpu as pltpu
from jax.experimental.pallas import tpu_sc as plsc

N = 10000
E = 320000
NP = 10240
EP = 327680
NSLICE = NP // 16
BIG = 60.0

_mesh = functools.partial(
    plsc.VectorSubcoreMesh, core_axis_name="c", subcore_axis_name="s")


def _bfs_body(src, dst, seed,
              m0, m1, m2, m3, m4,
              frontier_v, visited_v, reached_v, hops_v, zer_v,
              srcbs, dstbs, valbs, reached_sh, sem):
    cid = lax.axis_index("c")
    sid = lax.axis_index("s")
    slice_lo = sid * NSLICE
    mask_refs = (m0, m1, m2, m3)

    zvec = jnp.zeros((16,), jnp.float32)
    for k in range(NSLICE // 16):
        zer_v[pl.ds(k * 16, 16)] = zvec

    def zero_full(ref):
        def zbody(k, _):
            ref[pl.ds(k * 16, 16)] = zvec
            return 0
        lax.fori_loop(0, NP // 16, zbody, 0)

    zero_full(hops_v)
    pltpu.sync_copy(seed, frontier_v)
    pltpu.sync_copy(seed, visited_v)

    @pl.when(cid == 0)
    def _():
        pltpu.sync_copy(frontier_v.at[pl.ds(slice_lo, NSLICE)],
                        m0.at[pl.ds(slice_lo, NSLICE)])

    ebase0 = sid * (EP // 16)

    for hop in (1, 2, 3):
        pltpu.sync_copy(zer_v, reached_sh.at[pl.ds(slice_lo, NSLICE)])
        plsc.subcore_barrier()

        def chunk(j, _):
            eb = ebase0 + j * 1024
            ds = []
            for r in range(8):
                ds.append(pltpu.async_copy(
                    src.at[pl.ds(eb + r * 128, 128)], srcbs[r], sem))
                ds.append(pltpu.async_copy(
                    dst.at[pl.ds(eb + r * 128, 128)], dstbs[r], sem))
            for d in ds:
                d.wait()
            for r in range(8):
                for k in range(8):
                    idx = srcbs[r][pl.ds(k * 16, 16)]
                    v = plsc.load_gather(frontier_v, [idx])
                    valbs[r][pl.ds(k * 16, 16)] = v
            ds = []
            for r in range(8):
                ds.append(pltpu.async_copy(valbs[r], reached_sh.at[dstbs[r]],
                                           sem, add=True))
            for d in ds:
                d.wait()
            return 0

        lax.fori_loop(0, (EP // 16) // 1024, chunk, 0)
        plsc.subcore_barrier()

        pltpu.sync_copy(reached_sh, reached_v)
        hopf = jnp.float32(hop)

        def upd(k, _):
            sl = pl.ds(k * 16, 16)
            r = reached_v[sl]
            vis = visited_v[sl]
            nf = jnp.where((r > 0.0) & (vis < 0.5),
                           jnp.float32(1.0), jnp.float32(0.0))
            visited_v[sl] = vis + nf
            frontier_v[sl] = nf
            hops_v[sl] = hops_v[sl] + hopf * nf
            return 0

        lax.fori_loop(0, NP // 16, upd, 0)

        @pl.when(cid == 0)
        def _():
            pltpu.sync_copy(frontier_v.at[pl.ds(slice_lo, NSLICE)],
                            mask_refs[hop].at[pl.ds(slice_lo, NSLICE)])
        plsc.subcore_barrier()

    @pl.when(cid == 0)
    def _():
        pltpu.sync_copy(hops_v.at[pl.ds(slice_lo, NSLICE)],
                        m4.at[pl.ds(slice_lo, NSLICE)])


def _bfs(src, dst, seed):
    node = jax.ShapeDtypeStruct((NP,), jnp.float32)
    return pl.kernel(
        _bfs_body,
        out_type=(node,) * 5,
        mesh=_mesh(),
        scratch_types=[
            pltpu.VMEM((NP,), jnp.float32),
            pltpu.VMEM((NP,), jnp.float32),
            pltpu.VMEM((NP,), jnp.float32),
            pltpu.VMEM((NP,), jnp.float32),
            pltpu.VMEM((NSLICE,), jnp.float32),
            tuple(pltpu.VMEM((128,), jnp.int32) for _ in range(8)),
            tuple(pltpu.VMEM((128,), jnp.int32) for _ in range(8)),
            tuple(pltpu.VMEM((128,), jnp.float32) for _ in range(8)),
            pltpu.VMEM_SHARED((NP,), jnp.float32),
            pltpu.SemaphoreType.DMA,
        ],
        name="bfs_mask",
        compiler_params=pltpu.CompilerParams(needs_layout_passes=False, use_tc_tiling_on_sc=False),
    )(src, dst, seed)


def _make_gat_kernel(W, c_per_head, hoff=0):
    CE = 512
    CHUNKS = (EP // 32) // CE
    nk = W // 16

    def body(src, dst, asrc, adst, hmat, m16, zw, z16,
             outp0, outp1, denp0, denp1,
             srcb, dstb, sbuf, dbuf, hbuf, m_v,
             out_sh, den_sh, sem, sem_sc):
        cid = lax.axis_index("c")
        sid = lax.axis_index("s")
        wid = sid * 2 + cid
        slice_lo = sid * NSLICE

        pltpu.sync_copy(zw, out_sh.at[pl.ds(slice_lo, NSLICE)])
        pltpu.sync_copy(z16, den_sh.at[pl.ds(slice_lo, NSLICE)])
        pltpu.sync_copy(m16, m_v)
        plsc.subcore_barrier()

        ebase0 = wid * (EP // 32)
        iota = lax.iota(jnp.int32, 16)
        hidx = [(iota + k * 16) // c_per_head + hoff for k in range(nk)]

        def drain_scatters():
            pltpu.make_async_copy(z16.at[pl.ds(0, CE)], sbuf, sem_sc).wait()
            pltpu.make_async_copy(zw.at[pl.ds(0, CE)], hbuf, sem_sc).wait()

        def chunk(j, _):
            @pl.when(j > 0)
            def _():
                drain_scatters()
            eb = ebase0 + j * CE
            d1 = pltpu.async_copy(src.at[pl.ds(eb, CE)], srcb, sem)
            d2 = pltpu.async_copy(dst.at[pl.ds(eb, CE)], dstb, sem)
            d1.wait()
            d2.wait()
            g1 = pltpu.async_copy(asrc.at[srcb], sbuf, sem)
            g2 = pltpu.async_copy(adst.at[dstb], dbuf, sem)
            g3 = pltpu.async_copy(hmat.at[srcb], hbuf, sem)
            mv = m_v[...]
            g1.wait()
            g2.wait()

            def escore(i2, _):
                for u in range(4):
                    i = i2 * 4 + u
                    e = sbuf[i, :] + dbuf[i, :]
                    e = jnp.maximum(e, e * 0.2)
                    sbuf[i, :] = jnp.exp(e - mv)
                return 0

            lax.fori_loop(0, CE // 4, escore, 0)
            g3.wait()

            def emul(i2, _):
                for u in range(2):
                    i = i2 * 2 + u
                    ii = jnp.full((16,), i, jnp.int32)
                    for k in range(nk):
                        w = plsc.load_gather(sbuf, [ii, hidx[k]])
                        sl = pl.ds(k * 16, 16)
                        hbuf[i, sl] = hbuf[i, sl] * w
                return 0

            lax.fori_loop(0, CE // 2, emul, 0)
            pltpu.async_copy(sbuf, den_sh.at[dstb], sem_sc, add=True)
            pltpu.async_copy(hbuf, out_sh.at[dstb], sem_sc, add=True)
            return 0

        lax.fori_loop(0, CHUNKS, chunk, 0)
        drain_scatters()
        plsc.subcore_barrier()

        osl = pl.ds(slice_lo, NSLICE)

        @pl.when(cid == 0)
        def _():
            pltpu.sync_copy(out_sh.at[osl], outp0.at[osl])
            pltpu.sync_copy(den_sh.at[osl], denp0.at[osl])

        @pl.when(cid == 1)
        def _():
            pltpu.sync_copy(out_sh.at[osl], outp1.at[osl])
            pltpu.sync_copy(den_sh.at[osl], denp1.at[osl])

    def run(src, dst, asrc, adst, hmat, m16, zw, z16):
        return pl.kernel(
            body,
            out_type=(
                jax.ShapeDtypeStruct((NP, W), jnp.float32),
                jax.ShapeDtypeStruct((NP, W), jnp.float32),
                jax.ShapeDtypeStruct((NP, 16), jnp.float32),
                jax.ShapeDtypeStruct((NP, 16), jnp.float32),
            ),
            mesh=_mesh(),
            scratch_types=[
                pltpu.VMEM((512,), jnp.int32),
                pltpu.VMEM((512,), jnp.int32),
                pltpu.VMEM((512, 16), jnp.float32),
                pltpu.VMEM((512, 16), jnp.float32),
                pltpu.VMEM((512, W), jnp.float32),
                pltpu.VMEM((16,), jnp.float32),
                pltpu.VMEM_SHARED((NP, W), jnp.float32),
                pltpu.VMEM_SHARED((NP, 16), jnp.float32),
                pltpu.SemaphoreType.DMA,
                pltpu.SemaphoreType.DMA,
            ],
            name=f"gat_edges_w{W}_h{hoff}",
            compiler_params=pltpu.CompilerParams(needs_layout_passes=False, use_tc_tiling_on_sc=False),
        )(src, dst, asrc, adst, hmat, m16, zw, z16)

    return run


_gat_edges_64 = _make_gat_kernel(64, 8)
_gat_edges_l2lo = _make_gat_kernel(64, 16, 0)
_gat_edges_l2hi = _make_gat_kernel(64, 16, 4)
_gat_edges_l3 = _make_gat_kernel(64, 64)


BLK = 1024
GRID = NP // BLK


def _row_spec(w):
    return pl.BlockSpec((BLK, w), lambda i: (i, 0))


def _full_spec(r, w):
    return pl.BlockSpec((r, w), lambda i: (0, 0))


def _score_and_max(t, a_s, a_d, mxs_ref, mxd_ref, i):
    s = jnp.dot(t, a_s, preferred_element_type=jnp.float32)
    d = jnp.dot(t, a_d, preferred_element_type=jnp.float32)

    @pl.when(i == 0)
    def _():
        mxs_ref[...] = jnp.full((1, 16), -1e30, jnp.float32)
        mxd_ref[...] = jnp.full((1, 16), -1e30, jnp.float32)

    mxs_ref[...] = jnp.maximum(mxs_ref[...], jnp.max(s, axis=0, keepdims=True))
    mxd_ref[...] = jnp.maximum(mxd_ref[...], jnp.max(d, axis=0, keepdims=True))
    return s, d


def _ka_body(x_ref, hm_ref, w1x_ref, w1m_ref, a1s_ref, a1d_ref,
             t1_ref, s1_ref, d1_ref, mxs_ref, mxd_ref):
    i = pl.program_id(0)
    t1 = (jnp.dot(x_ref[...], w1x_ref[...], preferred_element_type=jnp.float32)
          + jnp.dot(hm_ref[...], w1m_ref[...],
                    preferred_element_type=jnp.float32))
    t1_ref[...] = t1
    s, d = _score_and_max(t1, a1s_ref[...], a1d_ref[...], mxs_ref, mxd_ref, i)
    s1_ref[...] = s
    d1_ref[...] = d


def _kernel_a(xp, hmp, w1x, w1m, a1s, a1d):
    return pl.pallas_call(
        _ka_body,
        grid=(GRID,),
        in_specs=[_row_spec(128), _row_spec(8), _full_spec(128, 64),
                  _full_spec(8, 64), _full_spec(64, 16), _full_spec(64, 16)],
        out_specs=[_row_spec(64), _row_spec(16), _row_spec(16),
                   _full_spec(1, 16), _full_spec(1, 16)],
        out_shape=[
            jax.ShapeDtypeStruct((NP, 64), jnp.float32),
            jax.ShapeDtypeStruct((NP, 16), jnp.float32),
            jax.ShapeDtypeStruct((NP, 16), jnp.float32),
            jax.ShapeDtypeStruct((1, 16), jnp.float32),
            jax.ShapeDtypeStruct((1, 16), jnp.float32),
        ],
    )(xp, hmp, w1x, w1m, a1s, a1d)


def _combine(o0, o1, d0, d1, t, s, d, m16, heads, c):
    sd = s + d
    ee_self = jnp.exp(jnp.where(sd > 0, sd, sd * 0.2) - m16)
    den = d0 + d1 + ee_self
    num = o0 + o1
    parts = []
    for hd in range(heads):
        nh = num[:, hd * c:(hd + 1) * c] + \
            t[:, hd * c:(hd + 1) * c] * ee_self[:, hd:hd + 1]
        parts.append(nh / den[:, hd:hd + 1])
    return jnp.concatenate(parts, axis=1)


def _kb1_body(o0_ref, o1_ref, dn0_ref, dn1_ref, t1_ref, s1_ref, d1_ref,
              m16_ref, b1_ref, w2_ref, a2s_ref, a2d_ref,
              t2_ref, s2_ref, d2_ref, mxs_ref, mxd_ref):
    i = pl.program_id(0)
    agg = _combine(o0_ref[...], o1_ref[...], dn0_ref[...], dn1_ref[...],
                   t1_ref[...], s1_ref[...], d1_ref[...], m16_ref[...], 8, 8)
    h1 = agg + b1_ref[...]
    h1 = jnp.where(h1 > 0, h1, jnp.exp(h1) - 1.0)
    t2 = jnp.dot(h1, w2_ref[...], preferred_element_type=jnp.float32)
    t2_ref[...] = t2
    s, d = _score_and_max(t2, a2s_ref[...], a2d_ref[...], mxs_ref, mxd_ref, i)
    s2_ref[...] = s
    d2_ref[...] = d


def _kernel_b1(o0, o1, dn0, dn1, t1, s1, d1, m16, b1, w2, a2s, a2d):
    return pl.pallas_call(
        _kb1_body,
        grid=(GRID,),
        in_specs=[
            _row_spec(64), _row_spec(64), _row_spec(16), _row_spec(16),
            _row_spec(64), _row_spec(16), _row_spec(16), _full_spec(1, 16),
            _full_spec(1, 64), _full_spec(64, 128),
            _full_spec(128, 16), _full_spec(128, 16)],
        out_specs=[_row_spec(128), _row_spec(16), _row_spec(16),
                   _full_spec(1, 16), _full_spec(1, 16)],
        out_shape=[
            jax.ShapeDtypeStruct((NP, 128), jnp.float32),
            jax.ShapeDtypeStruct((NP, 16), jnp.float32),
            jax.ShapeDtypeStruct((NP, 16), jnp.float32),
            jax.ShapeDtypeStruct((1, 16), jnp.float32),
            jax.ShapeDtypeStruct((1, 16), jnp.float32),
        ],
    )(o0, o1, dn0, dn1, t1, s1, d1, m16, b1, w2, a2s, a2d)


def _kb2_body(olo0_ref, olo1_ref, ohi0_ref, ohi1_ref,
              dlo0_ref, dlo1_ref, dhi0_ref, dhi1_ref,
              t2_ref, s2_ref, d2_ref,
              m16_ref, b2_ref, hop_ref, gwh_ref, gwp_ref, w3_ref,
              a3s_ref, a3d_ref,
              t3_ref, s3_ref, d3_ref, mxs_ref, mxd_ref):
    i = pl.program_id(0)
    sd = s2_ref[...] + d2_ref[...]
    ee_self = jnp.exp(jnp.where(sd > 0, sd, sd * 0.2) - m16_ref[...])
    den = (dlo0_ref[...] + dlo1_ref[...] + dhi0_ref[...] + dhi1_ref[...]
           + ee_self)
    nlo = olo0_ref[...] + olo1_ref[...]
    nhi = ohi0_ref[...] + ohi1_ref[...]
    t2v = t2_ref[...]
    parts = []
    for hd in range(8):
        base = nlo if hd < 4 else nhi
        nh = base[:, (hd % 4) * 16:(hd % 4 + 1) * 16] + \
            t2v[:, hd * 16:(hd + 1) * 16] * ee_self[:, hd:hd + 1]
        parts.append(nh / den[:, hd:hd + 1])
    h2 = jnp.concatenate(parts, axis=1) + b2_ref[...]
    g = jnp.dot(h2, gwh_ref[...], preferred_element_type=jnp.float32)[:, 0:1]
    g = g + hop_ref[:, 0:1] * gwp_ref[0, 0] + gwp_ref[0, 1]
    gate = 1.0 / (1.0 + jnp.exp(-g))
    h2g = h2 * gate
    t3 = jnp.dot(h2g, w3_ref[...], preferred_element_type=jnp.float32)
    t3_ref[...] = t3
    s, d = _score_and_max(t3, a3s_ref[...], a3d_ref[...], mxs_ref, mxd_ref, i)
    s3_ref[...] = s
    d3_ref[...] = d


def _kernel_b2(olo0, olo1, ohi0, ohi1, dlo0, dlo1, dhi0, dhi1,
               t2, s2, d2, m16, b2, hop, gwh, gwp, w3, a3s, a3d):
    return pl.pallas_call(
        _kb2_body,
        grid=(GRID,),
        in_specs=[
            _row_spec(64), _row_spec(64), _row_spec(64), _row_spec(64),
            _row_spec(16), _row_spec(16), _row_spec(16), _row_spec(16),
            _row_spec(128), _row_spec(16), _row_spec(16), _full_spec(1, 16),
            _full_spec(1, 128), _row_spec(16), _full_spec(128, 16),
            _full_spec(1, 16), _full_spec(128, 64),
            _full_spec(64, 16), _full_spec(64, 16)],
        out_specs=[_row_spec(64), _row_spec(16), _row_spec(16),
                   _full_spec(1, 16), _full_spec(1, 16)],
        out_shape=[
            jax.ShapeDtypeStruct((NP, 64), jnp.float32),
            jax.ShapeDtypeStruct((NP, 16), jnp.float32),
            jax.ShapeDtypeStruct((NP, 16), jnp.float32),
            jax.ShapeDtypeStruct((1, 16), jnp.float32),
            jax.ShapeDtypeStruct((1, 16), jnp.float32),
        ],
    )(olo0, olo1, ohi0, ohi1, dlo0, dlo1, dhi0, dhi1,
      t2, s2, d2, m16, b2, hop, gwh, gwp, w3, a3s, a3d)


def _kb3_body(o0_ref, o1_ref, dn0_ref, dn1_ref, t3_ref, s3_ref, d3_ref,
              m16_ref, b3_ref, out_ref):
    sd = s3_ref[...] + d3_ref[...]
    ee_self = jnp.exp(jnp.where(sd > 0, sd, sd * 0.2) - m16_ref[...])[:, 0:1]
    den = dn0_ref[...][:, 0:1] + dn1_ref[...][:, 0:1] + ee_self
    num = o0_ref[...] + o1_ref[...] + t3_ref[...] * ee_self
    h3 = num / den + b3_ref[...]
    logits = h3[:, 0:40]
    m = jnp.max(logits, axis=1, keepdims=True)
    lse = jnp.log(jnp.sum(jnp.exp(logits - m), axis=1, keepdims=True))
    out_ref[...] = logits - m - lse


def _kernel_b3(o0, o1, dn0, dn1, t3, s3, d3, m16, b3):
    return pl.pallas_call(
        _kb3_body,
        grid=(GRID,),
        in_specs=[
            _row_spec(64), _row_spec(64), _row_spec(16), _row_spec(16),
            _row_spec(64), _row_spec(16), _row_spec(16), _full_spec(1, 16),
            _full_spec(1, 64)],
        out_specs=[_row_spec(40)],
        out_shape=[jax.ShapeDtypeStruct((NP, 40), jnp.float32)],
    )(o0, o1, dn0, dn1, t3, s3, d3, m16, b3)[0]


def _block_diag16(a):
    heads, c = a.shape
    out = jnp.zeros((heads * c, 16), jnp.float32)
    rows = jnp.arange(heads * c)
    cols = jnp.repeat(jnp.arange(heads), c)
    return out.at[rows, cols].set(a.reshape(-1))


def _m16(mxs, mxd, heads):
    sd = mxs[0] + mxd[0]
    sd = jnp.where(sd > 0, sd, sd * 0.2)
    return jnp.where(jnp.arange(16) < heads, sd, BIG).astype(jnp.float32)


def kernel(x, edge_index, seed_mask, W1, a_src1, a_dst1, b1,
           W2, a_src2, a_dst2, b2, W3, a_src3, a_dst3, b3, gate_W, gate_b):
    f32 = jnp.float32
    src = jnp.concatenate([edge_index[0], jnp.full((EP - E,), N, jnp.int32)])
    dst = jnp.concatenate([edge_index[1], jnp.full((EP - E,), N, jnp.int32)])

    seed = jnp.pad(seed_mask.astype(f32), (0, NP - N))
    xp = jnp.pad(x, ((0, NP - N), (0, 0)))

    w1x = W1[:128]
    w1m = jnp.pad(W1[128:132], ((0, 4), (0, 0)))
    a1s, a1d = _block_diag16(a_src1), _block_diag16(a_dst1)
    a2s, a2d = _block_diag16(a_src2), _block_diag16(a_dst2)
    a3s = jnp.pad(_block_diag16(a_src3), ((0, 24), (0, 0)))
    a3d = jnp.pad(_block_diag16(a_dst3), ((0, 24), (0, 0)))
    w3p = jnp.pad(W3, ((0, 0), (0, 24)))
    b1r = b1[None, :]
    b2r = b2[None, :]
    b3r = jnp.pad(b3, (0, 24))[None, :]
    gwh = jnp.pad(gate_W[:128], ((0, 0), (0, 15)))
    gwp = jnp.pad(jnp.stack([gate_W[128], gate_b]).reshape(1, 2),
                  ((0, 0), (0, 14)))

    z64 = jnp.zeros((NSLICE, 64), f32)
    z16 = jnp.zeros((NSLICE, 16), f32)

    m0, m1, m2, m3, m4 = _bfs(src, dst, seed)
    hmp = jnp.stack([m0, m1, m2, m3], axis=1)
    hmp = jnp.pad(hmp, ((0, 0), (0, 4)))
    hop = m4[:, None] * jnp.ones((1, 16), f32)

    t1, s1, d1, mxs, mxd = _kernel_a(xp, hmp, w1x, w1m, a1s, a1d)
    m16_1 = _m16(mxs, mxd, 8)
    o0, o1, dn0, dn1 = _gat_edges_64(src, dst, s1, d1, t1, m16_1, z64, z16)
    t2, s2, d2, mxs2, mxd2 = _kernel_b1(
        o0, o1, dn0, dn1, t1, s1, d1, m16_1[None], b1r, W2, a2s, a2d)

    m16_2 = _m16(mxs2, mxd2, 8)
    ar = jnp.arange(16)
    m16_2lo = jnp.where(ar < 4, m16_2, BIG)
    m16_2hi = jnp.where((ar >= 4) & (ar < 8), m16_2, BIG)
    olo0, olo1, dlo0, dlo1 = _gat_edges_l2lo(
        src, dst, s2, d2, t2[:, :64], m16_2lo, z64, z16)
    ohi0, ohi1, dhi0, dhi1 = _gat_edges_l2hi(
        src, dst, s2, d2, t2[:, 64:], m16_2hi, z64, z16)
    t3, s3, d3, mxs3, mxd3 = _kernel_b2(
        olo0, olo1, ohi0, ohi1, dlo0, dlo1, dhi0, dhi1,
        t2, s2, d2, m16_2[None], b2r, hop, gwh, gwp, w3p, a3s, a3d)

    m16_3 = _m16(mxs3, mxd3, 1)
    o0, o1, dn0, dn1 = _gat_edges_l3(src, dst, s3, d3, t3, m16_3, z64, z16)
    o = _kernel_b3(o0, o1, dn0, dn1, t3, s3, d3, m16_3[None], b3r)
    return o[:N]


# speedup vs baseline: 1.3980x; 1.3980x over previous
import functools

import jax
import jax.numpy as jnp
from jax import lax
from jax.experimental import pallas as pl
from jax.experimental.pallas import tpu as pltpu
from jax.experimental.pallas import tpu_sc as plsc

N = 10000
E = 320000
NP = 10240
EP = 327680
NSLICE = NP // 16
BIG = 60.0

_mesh = functools.partial(
    plsc.VectorSubcoreMesh, core_axis_name="c", subcore_axis_name="s")


def _bfs_body(src, dst, seed,
              m0, m1, m2, m3, m4,
              frontier_v, visited_v, reached_v, hops_v, zer_v,
              srcbs, dstbs, valbs, reached_sh, sem):
    cid = lax.axis_index("c")
    sid = lax.axis_index("s")
    slice_lo = sid * NSLICE
    mask_refs = (m0, m1, m2, m3)

    zvec = jnp.zeros((16,), jnp.float32)
    for k in range(NSLICE // 16):
        zer_v[pl.ds(k * 16, 16)] = zvec

    def zero_full(ref):
        def zbody(k, _):
            ref[pl.ds(k * 16, 16)] = zvec
            return 0
        lax.fori_loop(0, NP // 16, zbody, 0)

    zero_full(hops_v)
    pltpu.sync_copy(seed, frontier_v)
    pltpu.sync_copy(seed, visited_v)

    @pl.when(cid == 0)
    def _():
        pltpu.sync_copy(frontier_v.at[pl.ds(slice_lo, NSLICE)],
                        m0.at[pl.ds(slice_lo, NSLICE)])

    ebase0 = sid * (EP // 16)

    for hop in (1, 2, 3):
        pltpu.sync_copy(zer_v, reached_sh.at[pl.ds(slice_lo, NSLICE)])
        plsc.subcore_barrier()

        def chunk(j, _):
            eb = ebase0 + j * 1024
            ds = []
            for r in range(8):
                ds.append(pltpu.async_copy(
                    src.at[pl.ds(eb + r * 128, 128)], srcbs[r], sem))
                ds.append(pltpu.async_copy(
                    dst.at[pl.ds(eb + r * 128, 128)], dstbs[r], sem))
            for d in ds:
                d.wait()
            for r in range(8):
                for k in range(8):
                    idx = srcbs[r][pl.ds(k * 16, 16)]
                    v = plsc.load_gather(frontier_v, [idx])
                    valbs[r][pl.ds(k * 16, 16)] = v
            ds = []
            for r in range(8):
                ds.append(pltpu.async_copy(valbs[r], reached_sh.at[dstbs[r]],
                                           sem, add=True))
            for d in ds:
                d.wait()
            return 0

        lax.fori_loop(0, (EP // 16) // 1024, chunk, 0)
        plsc.subcore_barrier()

        pltpu.sync_copy(reached_sh, reached_v)
        hopf = jnp.float32(hop)

        def upd(k, _):
            sl = pl.ds(k * 16, 16)
            r = reached_v[sl]
            vis = visited_v[sl]
            nf = jnp.where((r > 0.0) & (vis < 0.5),
                           jnp.float32(1.0), jnp.float32(0.0))
            visited_v[sl] = vis + nf
            frontier_v[sl] = nf
            hops_v[sl] = hops_v[sl] + hopf * nf
            return 0

        lax.fori_loop(0, NP // 16, upd, 0)

        @pl.when(cid == 0)
        def _():
            pltpu.sync_copy(frontier_v.at[pl.ds(slice_lo, NSLICE)],
                            mask_refs[hop].at[pl.ds(slice_lo, NSLICE)])
        plsc.subcore_barrier()

    @pl.when(cid == 0)
    def _():
        pltpu.sync_copy(hops_v.at[pl.ds(slice_lo, NSLICE)],
                        m4.at[pl.ds(slice_lo, NSLICE)])


def _bfs(src, dst, seed):
    node = jax.ShapeDtypeStruct((NP,), jnp.float32)
    return pl.kernel(
        _bfs_body,
        out_type=(node,) * 5,
        mesh=_mesh(),
        scratch_types=[
            pltpu.VMEM((NP,), jnp.float32),
            pltpu.VMEM((NP,), jnp.float32),
            pltpu.VMEM((NP,), jnp.float32),
            pltpu.VMEM((NP,), jnp.float32),
            pltpu.VMEM((NSLICE,), jnp.float32),
            tuple(pltpu.VMEM((128,), jnp.int32) for _ in range(8)),
            tuple(pltpu.VMEM((128,), jnp.int32) for _ in range(8)),
            tuple(pltpu.VMEM((128,), jnp.float32) for _ in range(8)),
            pltpu.VMEM_SHARED((NP,), jnp.float32),
            pltpu.SemaphoreType.DMA,
        ],
        name="bfs_mask",
        compiler_params=pltpu.CompilerParams(needs_layout_passes=False, use_tc_tiling_on_sc=False),
    )(src, dst, seed)


def _make_gat_kernel(W, c_per_head, hoff=0):
    CK = 4
    CHUNKS = (EP // 32) // (CK * 128)
    nk = W // 16

    def body(src, dst, asrc, adst, hmat, m16, zw, z16,
             outp0, outp1, denp0, denp1,
             srcbs, dstbs, sbufs, dbufs, hbufs, m_v,
             out_sh, den_sh, sem, sem_sc):
        cid = lax.axis_index("c")
        sid = lax.axis_index("s")
        wid = sid * 2 + cid
        slice_lo = sid * NSLICE

        pltpu.sync_copy(zw, out_sh.at[pl.ds(slice_lo, NSLICE)])
        pltpu.sync_copy(z16, den_sh.at[pl.ds(slice_lo, NSLICE)])
        pltpu.sync_copy(m16, m_v)
        plsc.subcore_barrier()

        ebase0 = wid * (EP // 32)
        iota = lax.iota(jnp.int32, 16)
        hidx = [(iota + k * 16) // c_per_head + hoff for k in range(nk)]

        def drain_scatters():
            for r in range(CK):
                pltpu.make_async_copy(z16.at[pl.ds(0, 128)], sbufs[r],
                                      sem_sc).wait()
                pltpu.make_async_copy(zw.at[pl.ds(0, 128)], hbufs[r],
                                      sem_sc).wait()

        def chunk(j, _):
            @pl.when(j > 0)
            def _():
                drain_scatters()
            eb = ebase0 + j * (CK * 128)
            ds = []
            for r in range(CK):
                ds.append(pltpu.async_copy(
                    src.at[pl.ds(eb + r * 128, 128)], srcbs[r], sem))
                ds.append(pltpu.async_copy(
                    dst.at[pl.ds(eb + r * 128, 128)], dstbs[r], sem))
            for d in ds:
                d.wait()
            gd = [None] * CK

            def fire(r):
                gd[r] = [
                    pltpu.async_copy(asrc.at[srcbs[r]], sbufs[r], sem),
                    pltpu.async_copy(adst.at[dstbs[r]], dbufs[r], sem),
                    pltpu.async_copy(hmat.at[srcbs[r]], hbufs[r], sem),
                ]

            fire(0)
            if CK > 1:
                fire(1)
            mv = m_v[...]
            for r in range(CK):
                for d in gd[r]:
                    d.wait()
                if r + 2 < CK:
                    fire(r + 2)
                sb, db, hb = sbufs[r], dbufs[r], hbufs[r]

                def escore(i2, _):
                    for u in range(4):
                        i = i2 * 4 + u
                        e = sb[i, :] + db[i, :]
                        e = jnp.maximum(e, e * 0.2)
                        sb[i, :] = jnp.exp(e - mv)
                    return 0

                lax.fori_loop(0, 32, escore, 0)

                def emul(i2, _):
                    for u in range(2):
                        i = i2 * 2 + u
                        ii = jnp.full((16,), i, jnp.int32)
                        for k in range(nk):
                            w = plsc.load_gather(sb, [ii, hidx[k]])
                            sl = pl.ds(k * 16, 16)
                            hb[i, sl] = hb[i, sl] * w
                    return 0

                lax.fori_loop(0, 64, emul, 0)
                pltpu.async_copy(sb, den_sh.at[dstbs[r]], sem_sc, add=True)
                pltpu.async_copy(hb, out_sh.at[dstbs[r]], sem_sc, add=True)
            return 0

        lax.fori_loop(0, CHUNKS, chunk, 0)
        drain_scatters()
        plsc.subcore_barrier()

        osl = pl.ds(slice_lo, NSLICE)

        @pl.when(cid == 0)
        def _():
            pltpu.sync_copy(out_sh.at[osl], outp0.at[osl])
            pltpu.sync_copy(den_sh.at[osl], denp0.at[osl])

        @pl.when(cid == 1)
        def _():
            pltpu.sync_copy(out_sh.at[osl], outp1.at[osl])
            pltpu.sync_copy(den_sh.at[osl], denp1.at[osl])

    def run(src, dst, asrc, adst, hmat, m16, zw, z16):
        return pl.kernel(
            body,
            out_type=(
                jax.ShapeDtypeStruct((NP, W), jnp.float32),
                jax.ShapeDtypeStruct((NP, W), jnp.float32),
                jax.ShapeDtypeStruct((NP, 16), jnp.float32),
                jax.ShapeDtypeStruct((NP, 16), jnp.float32),
            ),
            mesh=_mesh(),
            scratch_types=[
                tuple(pltpu.VMEM((128,), jnp.int32) for _ in range(CK)),
                tuple(pltpu.VMEM((128,), jnp.int32) for _ in range(CK)),
                tuple(pltpu.VMEM((128, 16), jnp.float32) for _ in range(CK)),
                tuple(pltpu.VMEM((128, 16), jnp.float32) for _ in range(CK)),
                tuple(pltpu.VMEM((128, W), jnp.float32) for _ in range(CK)),
                pltpu.VMEM((16,), jnp.float32),
                pltpu.VMEM_SHARED((NP, W), jnp.float32),
                pltpu.VMEM_SHARED((NP, 16), jnp.float32),
                pltpu.SemaphoreType.DMA,
                pltpu.SemaphoreType.DMA,
            ],
            name=f"gat_edges_w{W}_h{hoff}",
            compiler_params=pltpu.CompilerParams(needs_layout_passes=False, use_tc_tiling_on_sc=False),
        )(src, dst, asrc, adst, hmat, m16, zw, z16)

    return run


_gat_edges_64 = _make_gat_kernel(64, 8)
_gat_edges_l2lo = _make_gat_kernel(64, 16, 0)
_gat_edges_l2hi = _make_gat_kernel(64, 16, 4)
_gat_edges_l3 = _make_gat_kernel(64, 64)


BLK = 1024
GRID = NP // BLK


def _row_spec(w):
    return pl.BlockSpec((BLK, w), lambda i: (i, 0))


def _full_spec(r, w):
    return pl.BlockSpec((r, w), lambda i: (0, 0))


def _score_and_max(t, a_s, a_d, mxs_ref, mxd_ref, i):
    s = jnp.dot(t, a_s, preferred_element_type=jnp.float32)
    d = jnp.dot(t, a_d, preferred_element_type=jnp.float32)

    @pl.when(i == 0)
    def _():
        mxs_ref[...] = jnp.full((1, 16), -1e30, jnp.float32)
        mxd_ref[...] = jnp.full((1, 16), -1e30, jnp.float32)

    mxs_ref[...] = jnp.maximum(mxs_ref[...], jnp.max(s, axis=0, keepdims=True))
    mxd_ref[...] = jnp.maximum(mxd_ref[...], jnp.max(d, axis=0, keepdims=True))
    return s, d


def _ka_body(x_ref, hm_ref, w1x_ref, w1m_ref, a1s_ref, a1d_ref,
             t1_ref, s1_ref, d1_ref, mxs_ref, mxd_ref):
    i = pl.program_id(0)
    t1 = (jnp.dot(x_ref[...], w1x_ref[...], preferred_element_type=jnp.float32)
          + jnp.dot(hm_ref[...], w1m_ref[...],
                    preferred_element_type=jnp.float32))
    t1_ref[...] = t1
    s, d = _score_and_max(t1, a1s_ref[...], a1d_ref[...], mxs_ref, mxd_ref, i)
    s1_ref[...] = s
    d1_ref[...] = d


def _kernel_a(xp, hmp, w1x, w1m, a1s, a1d):
    return pl.pallas_call(
        _ka_body,
        grid=(GRID,),
        in_specs=[_row_spec(128), _row_spec(8), _full_spec(128, 64),
                  _full_spec(8, 64), _full_spec(64, 16), _full_spec(64, 16)],
        out_specs=[_row_spec(64), _row_spec(16), _row_spec(16),
                   _full_spec(1, 16), _full_spec(1, 16)],
        out_shape=[
            jax.ShapeDtypeStruct((NP, 64), jnp.float32),
            jax.ShapeDtypeStruct((NP, 16), jnp.float32),
            jax.ShapeDtypeStruct((NP, 16), jnp.float32),
            jax.ShapeDtypeStruct((1, 16), jnp.float32),
            jax.ShapeDtypeStruct((1, 16), jnp.float32),
        ],
    )(xp, hmp, w1x, w1m, a1s, a1d)


def _combine(o0, o1, d0, d1, t, s, d, m16, heads, c):
    sd = s + d
    ee_self = jnp.exp(jnp.where(sd > 0, sd, sd * 0.2) - m16)
    den = d0 + d1 + ee_self
    num = o0 + o1
    parts = []
    for hd in range(heads):
        nh = num[:, hd * c:(hd + 1) * c] + \
            t[:, hd * c:(hd + 1) * c] * ee_self[:, hd:hd + 1]
        parts.append(nh / den[:, hd:hd + 1])
    return jnp.concatenate(parts, axis=1)


def _kb1_body(o0_ref, o1_ref, dn0_ref, dn1_ref, t1_ref, s1_ref, d1_ref,
              m16_ref, b1_ref, w2_ref, a2s_ref, a2d_ref,
              t2_ref, s2_ref, d2_ref, mxs_ref, mxd_ref):
    i = pl.program_id(0)
    agg = _combine(o0_ref[...], o1_ref[...], dn0_ref[...], dn1_ref[...],
                   t1_ref[...], s1_ref[...], d1_ref[...], m16_ref[...], 8, 8)
    h1 = agg + b1_ref[...]
    h1 = jnp.where(h1 > 0, h1, jnp.exp(h1) - 1.0)
    t2 = jnp.dot(h1, w2_ref[...], preferred_element_type=jnp.float32)
    t2_ref[...] = t2
    s, d = _score_and_max(t2, a2s_ref[...], a2d_ref[...], mxs_ref, mxd_ref, i)
    s2_ref[...] = s
    d2_ref[...] = d


def _kernel_b1(o0, o1, dn0, dn1, t1, s1, d1, m16, b1, w2, a2s, a2d):
    return pl.pallas_call(
        _kb1_body,
        grid=(GRID,),
        in_specs=[
            _row_spec(64), _row_spec(64), _row_spec(16), _row_spec(16),
            _row_spec(64), _row_spec(16), _row_spec(16), _full_spec(1, 16),
            _full_spec(1, 64), _full_spec(64, 128),
            _full_spec(128, 16), _full_spec(128, 16)],
        out_specs=[_row_spec(128), _row_spec(16), _row_spec(16),
                   _full_spec(1, 16), _full_spec(1, 16)],
        out_shape=[
            jax.ShapeDtypeStruct((NP, 128), jnp.float32),
            jax.ShapeDtypeStruct((NP, 16), jnp.float32),
            jax.ShapeDtypeStruct((NP, 16), jnp.float32),
            jax.ShapeDtypeStruct((1, 16), jnp.float32),
            jax.ShapeDtypeStruct((1, 16), jnp.float32),
        ],
    )(o0, o1, dn0, dn1, t1, s1, d1, m16, b1, w2, a2s, a2d)


def _kb2_body(olo0_ref, olo1_ref, ohi0_ref, ohi1_ref,
              dlo0_ref, dlo1_ref, dhi0_ref, dhi1_ref,
              t2_ref, s2_ref, d2_ref,
              m16_ref, b2_ref, hop_ref, gwh_ref, gwp_ref, w3_ref,
              a3s_ref, a3d_ref,
              t3_ref, s3_ref, d3_ref, mxs_ref, mxd_ref):
    i = pl.program_id(0)
    sd = s2_ref[...] + d2_ref[...]
    ee_self = jnp.exp(jnp.where(sd > 0, sd, sd * 0.2) - m16_ref[...])
    den = (dlo0_ref[...] + dlo1_ref[...] + dhi0_ref[...] + dhi1_ref[...]
           + ee_self)
    nlo = olo0_ref[...] + olo1_ref[...]
    nhi = ohi0_ref[...] + ohi1_ref[...]
    t2v = t2_ref[...]
    parts = []
    for hd in range(8):
        base = nlo if hd < 4 else nhi
        nh = base[:, (hd % 4) * 16:(hd % 4 + 1) * 16] + \
            t2v[:, hd * 16:(hd + 1) * 16] * ee_self[:, hd:hd + 1]
        parts.append(nh / den[:, hd:hd + 1])
    h2 = jnp.concatenate(parts, axis=1) + b2_ref[...]
    g = jnp.dot(h2, gwh_ref[...], preferred_element_type=jnp.float32)[:, 0:1]
    g = g + hop_ref[:, 0:1] * gwp_ref[0, 0] + gwp_ref[0, 1]
    gate = 1.0 / (1.0 + jnp.exp(-g))
    h2g = h2 * gate
    t3 = jnp.dot(h2g, w3_ref[...], preferred_element_type=jnp.float32)
    t3_ref[...] = t3
    s, d = _score_and_max(t3, a3s_ref[...], a3d_ref[...], mxs_ref, mxd_ref, i)
    s3_ref[...] = s
    d3_ref[...] = d


def _kernel_b2(olo0, olo1, ohi0, ohi1, dlo0, dlo1, dhi0, dhi1,
               t2, s2, d2, m16, b2, hop, gwh, gwp, w3, a3s, a3d):
    return pl.pallas_call(
        _kb2_body,
        grid=(GRID,),
        in_specs=[
            _row_spec(64), _row_spec(64), _row_spec(64), _row_spec(64),
            _row_spec(16), _row_spec(16), _row_spec(16), _row_spec(16),
            _row_spec(128), _row_spec(16), _row_spec(16), _full_spec(1, 16),
            _full_spec(1, 128), _row_spec(16), _full_spec(128, 16),
            _full_spec(1, 16), _full_spec(128, 64),
            _full_spec(64, 16), _full_spec(64, 16)],
        out_specs=[_row_spec(64), _row_spec(16), _row_spec(16),
                   _full_spec(1, 16), _full_spec(1, 16)],
        out_shape=[
            jax.ShapeDtypeStruct((NP, 64), jnp.float32),
            jax.ShapeDtypeStruct((NP, 16), jnp.float32),
            jax.ShapeDtypeStruct((NP, 16), jnp.float32),
            jax.ShapeDtypeStruct((1, 16), jnp.float32),
            jax.ShapeDtypeStruct((1, 16), jnp.float32),
        ],
    )(olo0, olo1, ohi0, ohi1, dlo0, dlo1, dhi0, dhi1,
      t2, s2, d2, m16, b2, hop, gwh, gwp, w3, a3s, a3d)


def _kb3_body(o0_ref, o1_ref, dn0_ref, dn1_ref, t3_ref, s3_ref, d3_ref,
              m16_ref, b3_ref, out_ref):
    sd = s3_ref[...] + d3_ref[...]
    ee_self = jnp.exp(jnp.where(sd > 0, sd, sd * 0.2) - m16_ref[...])[:, 0:1]
    den = dn0_ref[...][:, 0:1] + dn1_ref[...][:, 0:1] + ee_self
    num = o0_ref[...] + o1_ref[...] + t3_ref[...] * ee_self
    h3 = num / den + b3_ref[...]
    logits = h3[:, 0:40]
    m = jnp.max(logits, axis=1, keepdims=True)
    lse = jnp.log(jnp.sum(jnp.exp(logits - m), axis=1, keepdims=True))
    out_ref[...] = logits - m - lse


def _kernel_b3(o0, o1, dn0, dn1, t3, s3, d3, m16, b3):
    return pl.pallas_call(
        _kb3_body,
        grid=(GRID,),
        in_specs=[
            _row_spec(64), _row_spec(64), _row_spec(16), _row_spec(16),
            _row_spec(64), _row_spec(16), _row_spec(16), _full_spec(1, 16),
            _full_spec(1, 64)],
        out_specs=[_row_spec(40)],
        out_shape=[jax.ShapeDtypeStruct((NP, 40), jnp.float32)],
    )(o0, o1, dn0, dn1, t3, s3, d3, m16, b3)[0]


def _block_diag16(a):
    heads, c = a.shape
    out = jnp.zeros((heads * c, 16), jnp.float32)
    rows = jnp.arange(heads * c)
    cols = jnp.repeat(jnp.arange(heads), c)
    return out.at[rows, cols].set(a.reshape(-1))


def _m16(mxs, mxd, heads):
    sd = mxs[0] + mxd[0]
    sd = jnp.where(sd > 0, sd, sd * 0.2)
    return jnp.where(jnp.arange(16) < heads, sd, BIG).astype(jnp.float32)


def kernel(x, edge_index, seed_mask, W1, a_src1, a_dst1, b1,
           W2, a_src2, a_dst2, b2, W3, a_src3, a_dst3, b3, gate_W, gate_b):
    f32 = jnp.float32
    src = jnp.concatenate([edge_index[0], jnp.full((EP - E,), N, jnp.int32)])
    dst = jnp.concatenate([edge_index[1], jnp.full((EP - E,), N, jnp.int32)])

    seed = jnp.pad(seed_mask.astype(f32), (0, NP - N))
    xp = jnp.pad(x, ((0, NP - N), (0, 0)))

    w1x = W1[:128]
    w1m = jnp.pad(W1[128:132], ((0, 4), (0, 0)))
    a1s, a1d = _block_diag16(a_src1), _block_diag16(a_dst1)
    a2s, a2d = _block_diag16(a_src2), _block_diag16(a_dst2)
    a3s = jnp.pad(_block_diag16(a_src3), ((0, 24), (0, 0)))
    a3d = jnp.pad(_block_diag16(a_dst3), ((0, 24), (0, 0)))
    w3p = jnp.pad(W3, ((0, 0), (0, 24)))
    b1r = b1[None, :]
    b2r = b2[None, :]
    b3r = jnp.pad(b3, (0, 24))[None, :]
    gwh = jnp.pad(gate_W[:128], ((0, 0), (0, 15)))
    gwp = jnp.pad(jnp.stack([gate_W[128], gate_b]).reshape(1, 2),
                  ((0, 0), (0, 14)))

    z64 = jnp.zeros((NSLICE, 64), f32)
    z16 = jnp.zeros((NSLICE, 16), f32)

    m0, m1, m2, m3, m4 = _bfs(src, dst, seed)
    hmp = jnp.stack([m0, m1, m2, m3], axis=1)
    hmp = jnp.pad(hmp, ((0, 0), (0, 4)))
    hop = m4[:, None] * jnp.ones((1, 16), f32)

    t1, s1, d1, mxs, mxd = _kernel_a(xp, hmp, w1x, w1m, a1s, a1d)
    m16_1 = _m16(mxs, mxd, 8)
    o0, o1, dn0, dn1 = _gat_edges_64(src, dst, s1, d1, t1, m16_1, z64, z16)
    t2, s2, d2, mxs2, mxd2 = _kernel_b1(
        o0, o1, dn0, dn1, t1, s1, d1, m16_1[None], b1r, W2, a2s, a2d)

    m16_2 = _m16(mxs2, mxd2, 8)
    ar = jnp.arange(16)
    m16_2lo = jnp.where(ar < 4, m16_2, BIG)
    m16_2hi = jnp.where((ar >= 4) & (ar < 8), m16_2, BIG)
    olo0, olo1, dlo0, dlo1 = _gat_edges_l2lo(
        src, dst, s2, d2, t2[:, :64], m16_2lo, z64, z16)
    ohi0, ohi1, dhi0, dhi1 = _gat_edges_l2hi(
        src, dst, s2, d2, t2[:, 64:], m16_2hi, z64, z16)
    t3, s3, d3, mxs3, mxd3 = _kernel_b2(
        olo0, olo1, ohi0, ohi1, dlo0, dlo1, dhi0, dhi1,
        t2, s2, d2, m16_2[None], b2r, hop, gwh, gwp, w3p, a3s, a3d)

    m16_3 = _m16(mxs3, mxd3, 1)
    o0, o1, dn0, dn1 = _gat_edges_l3(src, dst, s3, d3, t3, m16_3, z64, z16)
    o = _kernel_b3(o0, o1, dn0, dn1, t3, s3, d3, m16_3[None], b3r)
    return o[:N]


# speedup vs baseline: 1.4441x; 1.0330x over previous
import functools

import jax
import jax.numpy as jnp
from jax import lax
from jax.experimental import pallas as pl
from jax.experimental.pallas import tpu as pltpu
from jax.experimental.pallas import tpu_sc as plsc

N = 10000
E = 320000
NP = 10240
EP = 327680
NSLICE = NP // 16
BIG = 60.0

_mesh = functools.partial(
    plsc.VectorSubcoreMesh, core_axis_name="c", subcore_axis_name="s")


def _bfs_body(src, dst, seed,
              m0, m1, m2, m3, m4,
              frontier_v, visited_v, reached_v, hops_v, zer_v,
              srcbs, dstbs, valbs, reached_sh, sem):
    cid = lax.axis_index("c")
    sid = lax.axis_index("s")
    slice_lo = sid * NSLICE
    mask_refs = (m0, m1, m2, m3)

    zvec = jnp.zeros((16,), jnp.float32)
    for k in range(NSLICE // 16):
        zer_v[pl.ds(k * 16, 16)] = zvec

    def zero_full(ref):
        def zbody(k, _):
            ref[pl.ds(k * 16, 16)] = zvec
            return 0
        lax.fori_loop(0, NP // 16, zbody, 0)

    zero_full(hops_v)
    pltpu.sync_copy(seed, frontier_v)
    pltpu.sync_copy(seed, visited_v)

    @pl.when(cid == 0)
    def _():
        pltpu.sync_copy(frontier_v.at[pl.ds(slice_lo, NSLICE)],
                        m0.at[pl.ds(slice_lo, NSLICE)])

    ebase0 = sid * (EP // 16)

    for hop in (1, 2, 3):
        pltpu.sync_copy(zer_v, reached_sh.at[pl.ds(slice_lo, NSLICE)])
        plsc.subcore_barrier()

        def chunk(j, _):
            eb = ebase0 + j * 1024
            ds = []
            for r in range(8):
                ds.append(pltpu.async_copy(
                    src.at[pl.ds(eb + r * 128, 128)], srcbs[r], sem))
                ds.append(pltpu.async_copy(
                    dst.at[pl.ds(eb + r * 128, 128)], dstbs[r], sem))
            for d in ds:
                d.wait()
            for r in range(8):
                for k in range(8):
                    idx = srcbs[r][pl.ds(k * 16, 16)]
                    v = plsc.load_gather(frontier_v, [idx])
                    valbs[r][pl.ds(k * 16, 16)] = v
            ds = []
            for r in range(8):
                ds.append(pltpu.async_copy(valbs[r], reached_sh.at[dstbs[r]],
                                           sem, add=True))
            for d in ds:
                d.wait()
            return 0

        lax.fori_loop(0, (EP // 16) // 1024, chunk, 0)
        plsc.subcore_barrier()

        pltpu.sync_copy(reached_sh, reached_v)
        hopf = jnp.float32(hop)

        def upd(k, _):
            sl = pl.ds(k * 16, 16)
            r = reached_v[sl]
            vis = visited_v[sl]
            nf = jnp.where((r > 0.0) & (vis < 0.5),
                           jnp.float32(1.0), jnp.float32(0.0))
            visited_v[sl] = vis + nf
            frontier_v[sl] = nf
            hops_v[sl] = hops_v[sl] + hopf * nf
            return 0

        lax.fori_loop(0, NP // 16, upd, 0)

        @pl.when(cid == 0)
        def _():
            pltpu.sync_copy(frontier_v.at[pl.ds(slice_lo, NSLICE)],
                            mask_refs[hop].at[pl.ds(slice_lo, NSLICE)])
        plsc.subcore_barrier()

    @pl.when(cid == 0)
    def _():
        pltpu.sync_copy(hops_v.at[pl.ds(slice_lo, NSLICE)],
                        m4.at[pl.ds(slice_lo, NSLICE)])


def _bfs(src, dst, seed):
    node = jax.ShapeDtypeStruct((NP,), jnp.float32)
    return pl.kernel(
        _bfs_body,
        out_type=(node,) * 5,
        mesh=_mesh(),
        scratch_types=[
            pltpu.VMEM((NP,), jnp.float32),
            pltpu.VMEM((NP,), jnp.float32),
            pltpu.VMEM((NP,), jnp.float32),
            pltpu.VMEM((NP,), jnp.float32),
            pltpu.VMEM((NSLICE,), jnp.float32),
            tuple(pltpu.VMEM((128,), jnp.int32) for _ in range(8)),
            tuple(pltpu.VMEM((128,), jnp.int32) for _ in range(8)),
            tuple(pltpu.VMEM((128,), jnp.float32) for _ in range(8)),
            pltpu.VMEM_SHARED((NP,), jnp.float32),
            pltpu.SemaphoreType.DMA,
        ],
        name="bfs_mask",
        compiler_params=pltpu.CompilerParams(needs_layout_passes=False, use_tc_tiling_on_sc=False),
    )(src, dst, seed)


def _make_gat_kernel(W, c_per_head, hoff=0):
    CK = 5
    CHUNKS = (EP // 32) // (CK * 128)
    nk = W // 16

    def body(src, dst, asrc, adst, hmat, m16, zw, z16,
             outp0, outp1, denp0, denp1,
             srcbs, dstbs, sbufs, dbufs, hbufs, m_v,
             out_sh, den_sh, sem, sem_sc):
        cid = lax.axis_index("c")
        sid = lax.axis_index("s")
        wid = sid * 2 + cid
        slice_lo = sid * NSLICE

        pltpu.sync_copy(zw, out_sh.at[pl.ds(slice_lo, NSLICE)])
        pltpu.sync_copy(z16, den_sh.at[pl.ds(slice_lo, NSLICE)])
        pltpu.sync_copy(m16, m_v)
        plsc.subcore_barrier()

        ebase0 = wid * (EP // 32)
        iota = lax.iota(jnp.int32, 16)
        hidx = [(iota + k * 16) // c_per_head + hoff for k in range(nk)]

        def drain_scatters():
            for r in range(CK):
                pltpu.make_async_copy(z16.at[pl.ds(0, 128)], sbufs[r],
                                      sem_sc).wait()
                pltpu.make_async_copy(zw.at[pl.ds(0, 128)], hbufs[r],
                                      sem_sc).wait()

        def chunk(j, _):
            @pl.when(j > 0)
            def _():
                drain_scatters()
            eb = ebase0 + j * (CK * 128)
            ds = []
            for r in range(CK):
                ds.append(pltpu.async_copy(
                    src.at[pl.ds(eb + r * 128, 128)], srcbs[r], sem))
                ds.append(pltpu.async_copy(
                    dst.at[pl.ds(eb + r * 128, 128)], dstbs[r], sem))
            for d in ds:
                d.wait()
            gd = [None] * CK

            def fire(r):
                gd[r] = [
                    pltpu.async_copy(asrc.at[srcbs[r]], sbufs[r], sem),
                    pltpu.async_copy(adst.at[dstbs[r]], dbufs[r], sem),
                    pltpu.async_copy(hmat.at[srcbs[r]], hbufs[r], sem),
                ]

            fire(0)
            if CK > 1:
                fire(1)
            mv = m_v[...]
            for r in range(CK):
                for d in gd[r]:
                    d.wait()
                if r + 2 < CK:
                    fire(r + 2)
                sb, db, hb = sbufs[r], dbufs[r], hbufs[r]

                def escore(i2, _):
                    for u in range(4):
                        i = i2 * 4 + u
                        e = sb[i, :] + db[i, :]
                        e = jnp.maximum(e, e * 0.2)
                        sb[i, :] = jnp.exp(e - mv)
                    return 0

                lax.fori_loop(0, 32, escore, 0)

                def emul(i2, _):
                    for u in range(2):
                        i = i2 * 2 + u
                        ii = jnp.full((16,), i, jnp.int32)
                        for k in range(nk):
                            w = plsc.load_gather(sb, [ii, hidx[k]])
                            sl = pl.ds(k * 16, 16)
                            hb[i, sl] = hb[i, sl] * w
                    return 0

                lax.fori_loop(0, 64, emul, 0)
                pltpu.async_copy(sb, den_sh.at[dstbs[r]], sem_sc, add=True)
                pltpu.async_copy(hb, out_sh.at[dstbs[r]], sem_sc, add=True)
            return 0

        lax.fori_loop(0, CHUNKS, chunk, 0)
        drain_scatters()
        plsc.subcore_barrier()

        osl = pl.ds(slice_lo, NSLICE)

        @pl.when(cid == 0)
        def _():
            pltpu.sync_copy(out_sh.at[osl], outp0.at[osl])
            pltpu.sync_copy(den_sh.at[osl], denp0.at[osl])

        @pl.when(cid == 1)
        def _():
            pltpu.sync_copy(out_sh.at[osl], outp1.at[osl])
            pltpu.sync_copy(den_sh.at[osl], denp1.at[osl])

    def run(src, dst, asrc, adst, hmat, m16, zw, z16):
        return pl.kernel(
            body,
            out_type=(
                jax.ShapeDtypeStruct((NP, W), jnp.float32),
                jax.ShapeDtypeStruct((NP, W), jnp.float32),
                jax.ShapeDtypeStruct((NP, 16), jnp.float32),
                jax.ShapeDtypeStruct((NP, 16), jnp.float32),
            ),
            mesh=_mesh(),
            scratch_types=[
                tuple(pltpu.VMEM((128,), jnp.int32) for _ in range(CK)),
                tuple(pltpu.VMEM((128,), jnp.int32) for _ in range(CK)),
                tuple(pltpu.VMEM((128, 16), jnp.float32) for _ in range(CK)),
                tuple(pltpu.VMEM((128, 16), jnp.float32) for _ in range(CK)),
                tuple(pltpu.VMEM((128, W), jnp.float32) for _ in range(CK)),
                pltpu.VMEM((16,), jnp.float32),
                pltpu.VMEM_SHARED((NP, W), jnp.float32),
                pltpu.VMEM_SHARED((NP, 16), jnp.float32),
                pltpu.SemaphoreType.DMA,
                pltpu.SemaphoreType.DMA,
            ],
            name=f"gat_edges_w{W}_h{hoff}",
            compiler_params=pltpu.CompilerParams(needs_layout_passes=False, use_tc_tiling_on_sc=False),
        )(src, dst, asrc, adst, hmat, m16, zw, z16)

    return run


_gat_edges_64 = _make_gat_kernel(64, 8)
_gat_edges_l2lo = _make_gat_kernel(64, 16, 0)
_gat_edges_l2hi = _make_gat_kernel(64, 16, 4)
_gat_edges_l3 = _make_gat_kernel(64, 64)


BLK = 1024
GRID = NP // BLK


def _row_spec(w):
    return pl.BlockSpec((BLK, w), lambda i: (i, 0))


def _full_spec(r, w):
    return pl.BlockSpec((r, w), lambda i: (0, 0))


def _score_and_max(t, a_s, a_d, mxs_ref, mxd_ref, i):
    s = jnp.dot(t, a_s, preferred_element_type=jnp.float32)
    d = jnp.dot(t, a_d, preferred_element_type=jnp.float32)

    @pl.when(i == 0)
    def _():
        mxs_ref[...] = jnp.full((1, 16), -1e30, jnp.float32)
        mxd_ref[...] = jnp.full((1, 16), -1e30, jnp.float32)

    mxs_ref[...] = jnp.maximum(mxs_ref[...], jnp.max(s, axis=0, keepdims=True))
    mxd_ref[...] = jnp.maximum(mxd_ref[...], jnp.max(d, axis=0, keepdims=True))
    return s, d


def _ka_body(x_ref, hm_ref, w1x_ref, w1m_ref, a1s_ref, a1d_ref,
             t1_ref, s1_ref, d1_ref, mxs_ref, mxd_ref):
    i = pl.program_id(0)
    t1 = (jnp.dot(x_ref[...], w1x_ref[...], preferred_element_type=jnp.float32)
          + jnp.dot(hm_ref[...], w1m_ref[...],
                    preferred_element_type=jnp.float32))
    t1_ref[...] = t1
    s, d = _score_and_max(t1, a1s_ref[...], a1d_ref[...], mxs_ref, mxd_ref, i)
    s1_ref[...] = s
    d1_ref[...] = d


def _kernel_a(xp, hmp, w1x, w1m, a1s, a1d):
    return pl.pallas_call(
        _ka_body,
        grid=(GRID,),
        in_specs=[_row_spec(128), _row_spec(8), _full_spec(128, 64),
                  _full_spec(8, 64), _full_spec(64, 16), _full_spec(64, 16)],
        out_specs=[_row_spec(64), _row_spec(16), _row_spec(16),
                   _full_spec(1, 16), _full_spec(1, 16)],
        out_shape=[
            jax.ShapeDtypeStruct((NP, 64), jnp.float32),
            jax.ShapeDtypeStruct((NP, 16), jnp.float32),
            jax.ShapeDtypeStruct((NP, 16), jnp.float32),
            jax.ShapeDtypeStruct((1, 16), jnp.float32),
            jax.ShapeDtypeStruct((1, 16), jnp.float32),
        ],
    )(xp, hmp, w1x, w1m, a1s, a1d)


def _combine(o0, o1, d0, d1, t, s, d, m16, heads, c):
    sd = s + d
    ee_self = jnp.exp(jnp.where(sd > 0, sd, sd * 0.2) - m16)
    den = d0 + d1 + ee_self
    num = o0 + o1
    parts = []
    for hd in range(heads):
        nh = num[:, hd * c:(hd + 1) * c] + \
            t[:, hd * c:(hd + 1) * c] * ee_self[:, hd:hd + 1]
        parts.append(nh / den[:, hd:hd + 1])
    return jnp.concatenate(parts, axis=1)


def _kb1_body(o0_ref, o1_ref, dn0_ref, dn1_ref, t1_ref, s1_ref, d1_ref,
              m16_ref, b1_ref, w2_ref, a2s_ref, a2d_ref,
              t2_ref, s2_ref, d2_ref, mxs_ref, mxd_ref):
    i = pl.program_id(0)
    agg = _combine(o0_ref[...], o1_ref[...], dn0_ref[...], dn1_ref[...],
                   t1_ref[...], s1_ref[...], d1_ref[...], m16_ref[...], 8, 8)
    h1 = agg + b1_ref[...]
    h1 = jnp.where(h1 > 0, h1, jnp.exp(h1) - 1.0)
    t2 = jnp.dot(h1, w2_ref[...], preferred_element_type=jnp.float32)
    t2_ref[...] = t2
    s, d = _score_and_max(t2, a2s_ref[...], a2d_ref[...], mxs_ref, mxd_ref, i)
    s2_ref[...] = s
    d2_ref[...] = d


def _kernel_b1(o0, o1, dn0, dn1, t1, s1, d1, m16, b1, w2, a2s, a2d):
    return pl.pallas_call(
        _kb1_body,
        grid=(GRID,),
        in_specs=[
            _row_spec(64), _row_spec(64), _row_spec(16), _row_spec(16),
            _row_spec(64), _row_spec(16), _row_spec(16), _full_spec(1, 16),
            _full_spec(1, 64), _full_spec(64, 128),
            _full_spec(128, 16), _full_spec(128, 16)],
        out_specs=[_row_spec(128), _row_spec(16), _row_spec(16),
                   _full_spec(1, 16), _full_spec(1, 16)],
        out_shape=[
            jax.ShapeDtypeStruct((NP, 128), jnp.float32),
            jax.ShapeDtypeStruct((NP, 16), jnp.float32),
            jax.ShapeDtypeStruct((NP, 16), jnp.float32),
            jax.ShapeDtypeStruct((1, 16), jnp.float32),
            jax.ShapeDtypeStruct((1, 16), jnp.float32),
        ],
    )(o0, o1, dn0, dn1, t1, s1, d1, m16, b1, w2, a2s, a2d)


def _kb2_body(olo0_ref, olo1_ref, ohi0_ref, ohi1_ref,
              dlo0_ref, dlo1_ref, dhi0_ref, dhi1_ref,
              t2_ref, s2_ref, d2_ref,
              m16_ref, b2_ref, hop_ref, gwh_ref, gwp_ref, w3_ref,
              a3s_ref, a3d_ref,
              t3_ref, s3_ref, d3_ref, mxs_ref, mxd_ref):
    i = pl.program_id(0)
    sd = s2_ref[...] + d2_ref[...]
    ee_self = jnp.exp(jnp.where(sd > 0, sd, sd * 0.2) - m16_ref[...])
    den = (dlo0_ref[...] + dlo1_ref[...] + dhi0_ref[...] + dhi1_ref[...]
           + ee_self)
    nlo = olo0_ref[...] + olo1_ref[...]
    nhi = ohi0_ref[...] + ohi1_ref[...]
    t2v = t2_ref[...]
    parts = []
    for hd in range(8):
        base = nlo if hd < 4 else nhi
        nh = base[:, (hd % 4) * 16:(hd % 4 + 1) * 16] + \
            t2v[:, hd * 16:(hd + 1) * 16] * ee_self[:, hd:hd + 1]
        parts.append(nh / den[:, hd:hd + 1])
    h2 = jnp.concatenate(parts, axis=1) + b2_ref[...]
    g = jnp.dot(h2, gwh_ref[...], preferred_element_type=jnp.float32)[:, 0:1]
    g = g + hop_ref[:, 0:1] * gwp_ref[0, 0] + gwp_ref[0, 1]
    gate = 1.0 / (1.0 + jnp.exp(-g))
    h2g = h2 * gate
    t3 = jnp.dot(h2g, w3_ref[...], preferred_element_type=jnp.float32)
    t3_ref[...] = t3
    s, d = _score_and_max(t3, a3s_ref[...], a3d_ref[...], mxs_ref, mxd_ref, i)
    s3_ref[...] = s
    d3_ref[...] = d


def _kernel_b2(olo0, olo1, ohi0, ohi1, dlo0, dlo1, dhi0, dhi1,
               t2, s2, d2, m16, b2, hop, gwh, gwp, w3, a3s, a3d):
    return pl.pallas_call(
        _kb2_body,
        grid=(GRID,),
        in_specs=[
            _row_spec(64), _row_spec(64), _row_spec(64), _row_spec(64),
            _row_spec(16), _row_spec(16), _row_spec(16), _row_spec(16),
            _row_spec(128), _row_spec(16), _row_spec(16), _full_spec(1, 16),
            _full_spec(1, 128), _row_spec(16), _full_spec(128, 16),
            _full_spec(1, 16), _full_spec(128, 64),
            _full_spec(64, 16), _full_spec(64, 16)],
        out_specs=[_row_spec(64), _row_spec(16), _row_spec(16),
                   _full_spec(1, 16), _full_spec(1, 16)],
        out_shape=[
            jax.ShapeDtypeStruct((NP, 64), jnp.float32),
            jax.ShapeDtypeStruct((NP, 16), jnp.float32),
            jax.ShapeDtypeStruct((NP, 16), jnp.float32),
            jax.ShapeDtypeStruct((1, 16), jnp.float32),
            jax.ShapeDtypeStruct((1, 16), jnp.float32),
        ],
    )(olo0, olo1, ohi0, ohi1, dlo0, dlo1, dhi0, dhi1,
      t2, s2, d2, m16, b2, hop, gwh, gwp, w3, a3s, a3d)


def _kb3_body(o0_ref, o1_ref, dn0_ref, dn1_ref, t3_ref, s3_ref, d3_ref,
              m16_ref, b3_ref, out_ref):
    sd = s3_ref[...] + d3_ref[...]
    ee_self = jnp.exp(jnp.where(sd > 0, sd, sd * 0.2) - m16_ref[...])[:, 0:1]
    den = dn0_ref[...][:, 0:1] + dn1_ref[...][:, 0:1] + ee_self
    num = o0_ref[...] + o1_ref[...] + t3_ref[...] * ee_self
    h3 = num / den + b3_ref[...]
    logits = h3[:, 0:40]
    m = jnp.max(logits, axis=1, keepdims=True)
    lse = jnp.log(jnp.sum(jnp.exp(logits - m), axis=1, keepdims=True))
    out_ref[...] = logits - m - lse


def _kernel_b3(o0, o1, dn0, dn1, t3, s3, d3, m16, b3):
    return pl.pallas_call(
        _kb3_body,
        grid=(GRID,),
        in_specs=[
            _row_spec(64), _row_spec(64), _row_spec(16), _row_spec(16),
            _row_spec(64), _row_spec(16), _row_spec(16), _full_spec(1, 16),
            _full_spec(1, 64)],
        out_specs=[_row_spec(40)],
        out_shape=[jax.ShapeDtypeStruct((NP, 40), jnp.float32)],
    )(o0, o1, dn0, dn1, t3, s3, d3, m16, b3)[0]


def _block_diag16(a):
    heads, c = a.shape
    out = jnp.zeros((heads * c, 16), jnp.float32)
    rows = jnp.arange(heads * c)
    cols = jnp.repeat(jnp.arange(heads), c)
    return out.at[rows, cols].set(a.reshape(-1))


def _m16(mxs, mxd, heads):
    sd = mxs[0] + mxd[0]
    sd = jnp.where(sd > 0, sd, sd * 0.2)
    return jnp.where(jnp.arange(16) < heads, sd, BIG).astype(jnp.float32)


def kernel(x, edge_index, seed_mask, W1, a_src1, a_dst1, b1,
           W2, a_src2, a_dst2, b2, W3, a_src3, a_dst3, b3, gate_W, gate_b):
    f32 = jnp.float32
    src = jnp.concatenate([edge_index[0], jnp.full((EP - E,), N, jnp.int32)])
    dst = jnp.concatenate([edge_index[1], jnp.full((EP - E,), N, jnp.int32)])

    seed = jnp.pad(seed_mask.astype(f32), (0, NP - N))
    xp = jnp.pad(x, ((0, NP - N), (0, 0)))

    w1x = W1[:128]
    w1m = jnp.pad(W1[128:132], ((0, 4), (0, 0)))
    a1s, a1d = _block_diag16(a_src1), _block_diag16(a_dst1)
    a2s, a2d = _block_diag16(a_src2), _block_diag16(a_dst2)
    a3s = jnp.pad(_block_diag16(a_src3), ((0, 24), (0, 0)))
    a3d = jnp.pad(_block_diag16(a_dst3), ((0, 24), (0, 0)))
    w3p = jnp.pad(W3, ((0, 0), (0, 24)))
    b1r = b1[None, :]
    b2r = b2[None, :]
    b3r = jnp.pad(b3, (0, 24))[None, :]
    gwh = jnp.pad(gate_W[:128], ((0, 0), (0, 15)))
    gwp = jnp.pad(jnp.stack([gate_W[128], gate_b]).reshape(1, 2),
                  ((0, 0), (0, 14)))

    z64 = jnp.zeros((NSLICE, 64), f32)
    z16 = jnp.zeros((NSLICE, 16), f32)

    m0, m1, m2, m3, m4 = _bfs(src, dst, seed)
    hmp = jnp.stack([m0, m1, m2, m3], axis=1)
    hmp = jnp.pad(hmp, ((0, 0), (0, 4)))
    hop = m4[:, None] * jnp.ones((1, 16), f32)

    t1, s1, d1, mxs, mxd = _kernel_a(xp, hmp, w1x, w1m, a1s, a1d)
    m16_1 = _m16(mxs, mxd, 8)
    o0, o1, dn0, dn1 = _gat_edges_64(src, dst, s1, d1, t1, m16_1, z64, z16)
    t2, s2, d2, mxs2, mxd2 = _kernel_b1(
        o0, o1, dn0, dn1, t1, s1, d1, m16_1[None], b1r, W2, a2s, a2d)

    m16_2 = _m16(mxs2, mxd2, 8)
    ar = jnp.arange(16)
    m16_2lo = jnp.where(ar < 4, m16_2, BIG)
    m16_2hi = jnp.where((ar >= 4) & (ar < 8), m16_2, BIG)
    olo0, olo1, dlo0, dlo1 = _gat_edges_l2lo(
        src, dst, s2, d2, t2[:, :64], m16_2lo, z64, z16)
    ohi0, ohi1, dhi0, dhi1 = _gat_edges_l2hi(
        src, dst, s2, d2, t2[:, 64:], m16_2hi, z64, z16)
    t3, s3, d3, mxs3, mxd3 = _kernel_b2(
        olo0, olo1, ohi0, ohi1, dlo0, dlo1, dhi0, dhi1,
        t2, s2, d2, m16_2[None], b2r, hop, gwh, gwp, w3p, a3s, a3d)

    m16_3 = _m16(mxs3, mxd3, 1)
    o0, o1, dn0, dn1 = _gat_edges_l3(src, dst, s3, d3, t3, m16_3, z64, z16)
    o = _kernel_b3(o0, o1, dn0, dn1, t3, s3, d3, m16_3[None], b3r)
    return o[:N]


# speedup vs baseline: 1.5232x; 1.0548x over previous
import functools

import jax
import jax.numpy as jnp
from jax import lax
from jax.experimental import pallas as pl
from jax.experimental.pallas import tpu as pltpu
from jax.experimental.pallas import tpu_sc as plsc

N = 10000
E = 320000
NP = 10240
EP = 327680
NSLICE = NP // 16
BIG = 60.0

_mesh = functools.partial(
    plsc.VectorSubcoreMesh, core_axis_name="c", subcore_axis_name="s")


def _bfs_body(src, dst, seed,
              m0, m1, m2, m3, m4,
              frontier_v, visited_v, reached_v, hops_v, zer_v,
              srcbs, dstbs, valbs, reached_sh, sem):
    cid = lax.axis_index("c")
    sid = lax.axis_index("s")
    slice_lo = sid * NSLICE
    mask_refs = (m0, m1, m2, m3)

    zvec = jnp.zeros((16,), jnp.float32)
    for k in range(NSLICE // 16):
        zer_v[pl.ds(k * 16, 16)] = zvec

    def zero_full(ref):
        def zbody(k, _):
            ref[pl.ds(k * 16, 16)] = zvec
            return 0
        lax.fori_loop(0, NP // 16, zbody, 0)

    zero_full(hops_v)
    pltpu.sync_copy(seed, frontier_v)
    pltpu.sync_copy(seed, visited_v)

    @pl.when(cid == 0)
    def _():
        pltpu.sync_copy(frontier_v.at[pl.ds(slice_lo, NSLICE)],
                        m0.at[pl.ds(slice_lo, NSLICE)])

    ebase0 = sid * (EP // 16)

    for hop in (1, 2, 3):
        pltpu.sync_copy(zer_v, reached_sh.at[pl.ds(slice_lo, NSLICE)])
        plsc.subcore_barrier()

        def chunk(j, _):
            eb = ebase0 + j * 1024
            ds = []
            for r in range(8):
                ds.append(pltpu.async_copy(
                    src.at[pl.ds(eb + r * 128, 128)], srcbs[r], sem))
                ds.append(pltpu.async_copy(
                    dst.at[pl.ds(eb + r * 128, 128)], dstbs[r], sem))
            for d in ds:
                d.wait()
            for r in range(8):
                for k in range(8):
                    idx = srcbs[r][pl.ds(k * 16, 16)]
                    v = plsc.load_gather(frontier_v, [idx])
                    valbs[r][pl.ds(k * 16, 16)] = v
            ds = []
            for r in range(8):
                ds.append(pltpu.async_copy(valbs[r], reached_sh.at[dstbs[r]],
                                           sem, add=True))
            for d in ds:
                d.wait()
            return 0

        lax.fori_loop(0, (EP // 16) // 1024, chunk, 0)
        plsc.subcore_barrier()

        pltpu.sync_copy(reached_sh, reached_v)
        hopf = jnp.float32(hop)

        def upd(k, _):
            sl = pl.ds(k * 16, 16)
            r = reached_v[sl]
            vis = visited_v[sl]
            nf = jnp.where((r > 0.0) & (vis < 0.5),
                           jnp.float32(1.0), jnp.float32(0.0))
            visited_v[sl] = vis + nf
            frontier_v[sl] = nf
            hops_v[sl] = hops_v[sl] + hopf * nf
            return 0

        lax.fori_loop(0, NP // 16, upd, 0)

        @pl.when(cid == 0)
        def _():
            pltpu.sync_copy(frontier_v.at[pl.ds(slice_lo, NSLICE)],
                            mask_refs[hop].at[pl.ds(slice_lo, NSLICE)])
        plsc.subcore_barrier()

    @pl.when(cid == 0)
    def _():
        pltpu.sync_copy(hops_v.at[pl.ds(slice_lo, NSLICE)],
                        m4.at[pl.ds(slice_lo, NSLICE)])


def _bfs(src, dst, seed):
    node = jax.ShapeDtypeStruct((NP,), jnp.float32)
    return pl.kernel(
        _bfs_body,
        out_type=(node,) * 5,
        mesh=_mesh(),
        scratch_types=[
            pltpu.VMEM((NP,), jnp.float32),
            pltpu.VMEM((NP,), jnp.float32),
            pltpu.VMEM((NP,), jnp.float32),
            pltpu.VMEM((NP,), jnp.float32),
            pltpu.VMEM((NSLICE,), jnp.float32),
            tuple(pltpu.VMEM((128,), jnp.int32) for _ in range(8)),
            tuple(pltpu.VMEM((128,), jnp.int32) for _ in range(8)),
            tuple(pltpu.VMEM((128,), jnp.float32) for _ in range(8)),
            pltpu.VMEM_SHARED((NP,), jnp.float32),
            pltpu.SemaphoreType.DMA,
        ],
        name="bfs_mask",
        compiler_params=pltpu.CompilerParams(needs_layout_passes=False, use_tc_tiling_on_sc=False),
    )(src, dst, seed)


def _make_gat_kernel(W, c_per_head, hoff=0):
    CK = 5
    CHUNKS = (EP // 32) // (CK * 128)
    nk = W // 16

    def body(src, dst, asrc, adst, hmat, m16, zw, z16,
             outp0, outp1, denp0, denp1,
             srcbs, dstbs, sbufs, dbufs, hbufs, m_v,
             out_sh, den_sh, sem, sem_sc):
        cid = lax.axis_index("c")
        sid = lax.axis_index("s")
        wid = sid * 2 + cid
        slice_lo = sid * NSLICE

        pltpu.sync_copy(zw, out_sh.at[pl.ds(slice_lo, NSLICE)])
        pltpu.sync_copy(z16, den_sh.at[pl.ds(slice_lo, NSLICE)])
        pltpu.sync_copy(m16, m_v)
        plsc.subcore_barrier()

        ebase0 = wid * (EP // 32)
        iota = lax.iota(jnp.int32, 16)
        hidx = [(iota + k * 16) // c_per_head + hoff for k in range(nk)]

        def drain_scatters():
            for r in range(CK):
                pltpu.make_async_copy(z16.at[pl.ds(0, 128)], sbufs[r],
                                      sem_sc).wait()
                pltpu.make_async_copy(zw.at[pl.ds(0, 128)], hbufs[r],
                                      sem_sc).wait()

        def chunk(j, _):
            @pl.when(j > 0)
            def _():
                drain_scatters()
            eb = ebase0 + j * (CK * 128)
            ds = []
            for r in range(CK):
                ds.append(pltpu.async_copy(
                    src.at[pl.ds(eb + r * 128, 128)], srcbs[r], sem))
                ds.append(pltpu.async_copy(
                    dst.at[pl.ds(eb + r * 128, 128)], dstbs[r], sem))
            for d in ds:
                d.wait()
            gd = [None] * CK

            def fire(r):
                gd[r] = [
                    pltpu.async_copy(asrc.at[srcbs[r]], sbufs[r], sem),
                    pltpu.async_copy(adst.at[dstbs[r]], dbufs[r], sem),
                    pltpu.async_copy(hmat.at[srcbs[r]], hbufs[r], sem),
                ]

            fire(0)
            if CK > 1:
                fire(1)
            mv = m_v[...]
            for r in range(CK):
                for d in gd[r]:
                    d.wait()
                if r + 2 < CK:
                    fire(r + 2)
                sb, db, hb = sbufs[r], dbufs[r], hbufs[r]

                def escore(i2, _):
                    for u in range(4):
                        i = i2 * 4 + u
                        e = sb[i, :] + db[i, :]
                        e = jnp.maximum(e, e * 0.2)
                        sb[i, :] = jnp.exp(e - mv)
                    return 0

                lax.fori_loop(0, 32, escore, 0)

                def emul(i2, _):
                    for u in range(2):
                        i = i2 * 2 + u
                        ii = jnp.full((16,), i, jnp.int32)
                        for k in range(nk):
                            w = plsc.load_gather(sb, [ii, hidx[k]])
                            sl = pl.ds(k * 16, 16)
                            hb[i, sl] = hb[i, sl] * w
                    return 0

                lax.fori_loop(0, 64, emul, 0)
                pltpu.async_copy(sb, den_sh.at[dstbs[r]], sem_sc, add=True)
                pltpu.async_copy(hb, out_sh.at[dstbs[r]], sem_sc, add=True)
            return 0

        lax.fori_loop(0, CHUNKS, chunk, 0)
        drain_scatters()
        plsc.subcore_barrier()

        osl = pl.ds(slice_lo, NSLICE)

        @pl.when(cid == 0)
        def _():
            pltpu.sync_copy(out_sh.at[osl], outp0.at[osl])
            pltpu.sync_copy(den_sh.at[osl], denp0.at[osl])

        @pl.when(cid == 1)
        def _():
            pltpu.sync_copy(out_sh.at[osl], outp1.at[osl])
            pltpu.sync_copy(den_sh.at[osl], denp1.at[osl])

    def run(src, dst, asrc, adst, hmat, m16, zw, z16):
        return pl.kernel(
            body,
            out_type=(
                jax.ShapeDtypeStruct((NP, W), jnp.float32),
                jax.ShapeDtypeStruct((NP, W), jnp.float32),
                jax.ShapeDtypeStruct((NP, 16), jnp.float32),
                jax.ShapeDtypeStruct((NP, 16), jnp.float32),
            ),
            mesh=_mesh(),
            scratch_types=[
                tuple(pltpu.VMEM((128,), jnp.int32) for _ in range(CK)),
                tuple(pltpu.VMEM((128,), jnp.int32) for _ in range(CK)),
                tuple(pltpu.VMEM((128, 16), jnp.float32) for _ in range(CK)),
                tuple(pltpu.VMEM((128, 16), jnp.float32) for _ in range(CK)),
                tuple(pltpu.VMEM((128, W), jnp.float32) for _ in range(CK)),
                pltpu.VMEM((16,), jnp.float32),
                pltpu.VMEM_SHARED((NP, W), jnp.float32),
                pltpu.VMEM_SHARED((NP, 16), jnp.float32),
                pltpu.SemaphoreType.DMA,
                pltpu.SemaphoreType.DMA,
            ],
            name=f"gat_edges_w{W}_h{hoff}",
            compiler_params=pltpu.CompilerParams(needs_layout_passes=False, use_tc_tiling_on_sc=False),
        )(src, dst, asrc, adst, hmat, m16, zw, z16)

    return run


_gat_edges_64 = _make_gat_kernel(64, 8)
_gat_edges_l2lo = _make_gat_kernel(64, 16, 0)
_gat_edges_l2hi = _make_gat_kernel(64, 16, 4)
_gat_edges_l3 = _make_gat_kernel(48, 48)


BLK = 1024
GRID = NP // BLK


def _row_spec(w):
    return pl.BlockSpec((BLK, w), lambda i: (i, 0))


def _full_spec(r, w):
    return pl.BlockSpec((r, w), lambda i: (0, 0))


def _score_and_max(t, a_s, a_d, mxs_ref, mxd_ref, i):
    s = jnp.dot(t, a_s, preferred_element_type=jnp.float32)
    d = jnp.dot(t, a_d, preferred_element_type=jnp.float32)

    @pl.when(i == 0)
    def _():
        mxs_ref[...] = jnp.full((1, 16), -1e30, jnp.float32)
        mxd_ref[...] = jnp.full((1, 16), -1e30, jnp.float32)

    mxs_ref[...] = jnp.maximum(mxs_ref[...], jnp.max(s, axis=0, keepdims=True))
    mxd_ref[...] = jnp.maximum(mxd_ref[...], jnp.max(d, axis=0, keepdims=True))
    return s, d


def _ka_body(x_ref, hm_ref, w1x_ref, w1m_ref, a1s_ref, a1d_ref,
             t1_ref, s1_ref, d1_ref, mxs_ref, mxd_ref):
    i = pl.program_id(0)
    t1 = (jnp.dot(x_ref[...], w1x_ref[...], preferred_element_type=jnp.float32)
          + jnp.dot(hm_ref[...], w1m_ref[...],
                    preferred_element_type=jnp.float32))
    t1_ref[...] = t1
    s, d = _score_and_max(t1, a1s_ref[...], a1d_ref[...], mxs_ref, mxd_ref, i)
    s1_ref[...] = s
    d1_ref[...] = d


def _kernel_a(xp, hmp, w1x, w1m, a1s, a1d):
    return pl.pallas_call(
        _ka_body,
        grid=(GRID,),
        in_specs=[_row_spec(128), _row_spec(8), _full_spec(128, 64),
                  _full_spec(8, 64), _full_spec(64, 16), _full_spec(64, 16)],
        out_specs=[_row_spec(64), _row_spec(16), _row_spec(16),
                   _full_spec(1, 16), _full_spec(1, 16)],
        out_shape=[
            jax.ShapeDtypeStruct((NP, 64), jnp.float32),
            jax.ShapeDtypeStruct((NP, 16), jnp.float32),
            jax.ShapeDtypeStruct((NP, 16), jnp.float32),
            jax.ShapeDtypeStruct((1, 16), jnp.float32),
            jax.ShapeDtypeStruct((1, 16), jnp.float32),
        ],
    )(xp, hmp, w1x, w1m, a1s, a1d)


def _combine(o0, o1, d0, d1, t, s, d, m16, heads, c):
    sd = s + d
    ee_self = jnp.exp(jnp.where(sd > 0, sd, sd * 0.2) - m16)
    den = d0 + d1 + ee_self
    num = o0 + o1
    parts = []
    for hd in range(heads):
        nh = num[:, hd * c:(hd + 1) * c] + \
            t[:, hd * c:(hd + 1) * c] * ee_self[:, hd:hd + 1]
        parts.append(nh / den[:, hd:hd + 1])
    return jnp.concatenate(parts, axis=1)


def _kb1_body(o0_ref, o1_ref, dn0_ref, dn1_ref, t1_ref, s1_ref, d1_ref,
              m16_ref, b1_ref, w2_ref, a2s_ref, a2d_ref,
              t2_ref, s2_ref, d2_ref, mxs_ref, mxd_ref):
    i = pl.program_id(0)
    agg = _combine(o0_ref[...], o1_ref[...], dn0_ref[...], dn1_ref[...],
                   t1_ref[...], s1_ref[...], d1_ref[...], m16_ref[...], 8, 8)
    h1 = agg + b1_ref[...]
    h1 = jnp.where(h1 > 0, h1, jnp.exp(h1) - 1.0)
    t2 = jnp.dot(h1, w2_ref[...], preferred_element_type=jnp.float32)
    t2_ref[...] = t2
    s, d = _score_and_max(t2, a2s_ref[...], a2d_ref[...], mxs_ref, mxd_ref, i)
    s2_ref[...] = s
    d2_ref[...] = d


def _kernel_b1(o0, o1, dn0, dn1, t1, s1, d1, m16, b1, w2, a2s, a2d):
    return pl.pallas_call(
        _kb1_body,
        grid=(GRID,),
        in_specs=[
            _row_spec(64), _row_spec(64), _row_spec(16), _row_spec(16),
            _row_spec(64), _row_spec(16), _row_spec(16), _full_spec(1, 16),
            _full_spec(1, 64), _full_spec(64, 128),
            _full_spec(128, 16), _full_spec(128, 16)],
        out_specs=[_row_spec(128), _row_spec(16), _row_spec(16),
                   _full_spec(1, 16), _full_spec(1, 16)],
        out_shape=[
            jax.ShapeDtypeStruct((NP, 128), jnp.float32),
            jax.ShapeDtypeStruct((NP, 16), jnp.float32),
            jax.ShapeDtypeStruct((NP, 16), jnp.float32),
            jax.ShapeDtypeStruct((1, 16), jnp.float32),
            jax.ShapeDtypeStruct((1, 16), jnp.float32),
        ],
    )(o0, o1, dn0, dn1, t1, s1, d1, m16, b1, w2, a2s, a2d)


def _kb2_body(olo0_ref, olo1_ref, ohi0_ref, ohi1_ref,
              dlo0_ref, dlo1_ref, dhi0_ref, dhi1_ref,
              t2_ref, s2_ref, d2_ref,
              m16_ref, b2_ref, hop_ref, gwh_ref, gwp_ref, w3_ref,
              a3s_ref, a3d_ref,
              t3_ref, s3_ref, d3_ref, mxs_ref, mxd_ref):
    i = pl.program_id(0)
    sd = s2_ref[...] + d2_ref[...]
    ee_self = jnp.exp(jnp.where(sd > 0, sd, sd * 0.2) - m16_ref[...])
    den = (dlo0_ref[...] + dlo1_ref[...] + dhi0_ref[...] + dhi1_ref[...]
           + ee_self)
    nlo = olo0_ref[...] + olo1_ref[...]
    nhi = ohi0_ref[...] + ohi1_ref[...]
    t2v = t2_ref[...]
    parts = []
    for hd in range(8):
        base = nlo if hd < 4 else nhi
        nh = base[:, (hd % 4) * 16:(hd % 4 + 1) * 16] + \
            t2v[:, hd * 16:(hd + 1) * 16] * ee_self[:, hd:hd + 1]
        parts.append(nh / den[:, hd:hd + 1])
    h2 = jnp.concatenate(parts, axis=1) + b2_ref[...]
    g = jnp.dot(h2, gwh_ref[...], preferred_element_type=jnp.float32)[:, 0:1]
    g = g + hop_ref[:, 0:1] * gwp_ref[0, 0] + gwp_ref[0, 1]
    gate = 1.0 / (1.0 + jnp.exp(-g))
    h2g = h2 * gate
    t3 = jnp.dot(h2g, w3_ref[...], preferred_element_type=jnp.float32)
    t3_ref[...] = t3
    s, d = _score_and_max(t3, a3s_ref[...], a3d_ref[...], mxs_ref, mxd_ref, i)
    s3_ref[...] = s
    d3_ref[...] = d


def _kernel_b2(olo0, olo1, ohi0, ohi1, dlo0, dlo1, dhi0, dhi1,
               t2, s2, d2, m16, b2, hop, gwh, gwp, w3, a3s, a3d):
    return pl.pallas_call(
        _kb2_body,
        grid=(GRID,),
        in_specs=[
            _row_spec(64), _row_spec(64), _row_spec(64), _row_spec(64),
            _row_spec(16), _row_spec(16), _row_spec(16), _row_spec(16),
            _row_spec(128), _row_spec(16), _row_spec(16), _full_spec(1, 16),
            _full_spec(1, 128), _row_spec(16), _full_spec(128, 16),
            _full_spec(1, 16), _full_spec(128, 64),
            _full_spec(64, 16), _full_spec(64, 16)],
        out_specs=[_row_spec(64), _row_spec(16), _row_spec(16),
                   _full_spec(1, 16), _full_spec(1, 16)],
        out_shape=[
            jax.ShapeDtypeStruct((NP, 64), jnp.float32),
            jax.ShapeDtypeStruct((NP, 16), jnp.float32),
            jax.ShapeDtypeStruct((NP, 16), jnp.float32),
            jax.ShapeDtypeStruct((1, 16), jnp.float32),
            jax.ShapeDtypeStruct((1, 16), jnp.float32),
        ],
    )(olo0, olo1, ohi0, ohi1, dlo0, dlo1, dhi0, dhi1,
      t2, s2, d2, m16, b2, hop, gwh, gwp, w3, a3s, a3d)


def _kb3_body(o0_ref, o1_ref, dn0_ref, dn1_ref, t3_ref, s3_ref, d3_ref,
              m16_ref, b3_ref, out_ref):
    sd = s3_ref[...] + d3_ref[...]
    ee_self = jnp.exp(jnp.where(sd > 0, sd, sd * 0.2) - m16_ref[...])[:, 0:1]
    den = dn0_ref[...][:, 0:1] + dn1_ref[...][:, 0:1] + ee_self
    num = o0_ref[...] + o1_ref[...] + t3_ref[...][:, 0:48] * ee_self
    h3 = num / den + b3_ref[...]
    logits = h3[:, 0:40]
    m = jnp.max(logits, axis=1, keepdims=True)
    lse = jnp.log(jnp.sum(jnp.exp(logits - m), axis=1, keepdims=True))
    out_ref[...] = logits - m - lse


def _kernel_b3(o0, o1, dn0, dn1, t3, s3, d3, m16, b3):
    return pl.pallas_call(
        _kb3_body,
        grid=(GRID,),
        in_specs=[
            _row_spec(48), _row_spec(48), _row_spec(16), _row_spec(16),
            _row_spec(64), _row_spec(16), _row_spec(16), _full_spec(1, 16),
            _full_spec(1, 48)],
        out_specs=[_row_spec(40)],
        out_shape=[jax.ShapeDtypeStruct((NP, 40), jnp.float32)],
    )(o0, o1, dn0, dn1, t3, s3, d3, m16, b3)[0]


def _block_diag16(a):
    heads, c = a.shape
    out = jnp.zeros((heads * c, 16), jnp.float32)
    rows = jnp.arange(heads * c)
    cols = jnp.repeat(jnp.arange(heads), c)
    return out.at[rows, cols].set(a.reshape(-1))


def _m16(mxs, mxd, heads):
    sd = mxs[0] + mxd[0]
    sd = jnp.where(sd > 0, sd, sd * 0.2)
    return jnp.where(jnp.arange(16) < heads, sd, BIG).astype(jnp.float32)


def kernel(x, edge_index, seed_mask, W1, a_src1, a_dst1, b1,
           W2, a_src2, a_dst2, b2, W3, a_src3, a_dst3, b3, gate_W, gate_b):
    f32 = jnp.float32
    src = jnp.concatenate([edge_index[0], jnp.full((EP - E,), N, jnp.int32)])
    dst = jnp.concatenate([edge_index[1], jnp.full((EP - E,), N, jnp.int32)])

    seed = jnp.pad(seed_mask.astype(f32), (0, NP - N))
    xp = jnp.pad(x, ((0, NP - N), (0, 0)))

    w1x = W1[:128]
    w1m = jnp.pad(W1[128:132], ((0, 4), (0, 0)))
    a1s, a1d = _block_diag16(a_src1), _block_diag16(a_dst1)
    a2s, a2d = _block_diag16(a_src2), _block_diag16(a_dst2)
    a3s = jnp.pad(_block_diag16(a_src3), ((0, 24), (0, 0)))
    a3d = jnp.pad(_block_diag16(a_dst3), ((0, 24), (0, 0)))
    w3p = jnp.pad(W3, ((0, 0), (0, 24)))
    b1r = b1[None, :]
    b2r = b2[None, :]
    b3r = jnp.pad(b3, (0, 8))[None, :]
    gwh = jnp.pad(gate_W[:128], ((0, 0), (0, 15)))
    gwp = jnp.pad(jnp.stack([gate_W[128], gate_b]).reshape(1, 2),
                  ((0, 0), (0, 14)))

    z64 = jnp.zeros((NSLICE, 64), f32)
    z16 = jnp.zeros((NSLICE, 16), f32)
    z48 = jnp.zeros((NSLICE, 48), f32)

    m0, m1, m2, m3, m4 = _bfs(src, dst, seed)
    hmp = jnp.stack([m0, m1, m2, m3], axis=1)
    hmp = jnp.pad(hmp, ((0, 0), (0, 4)))
    hop = m4[:, None] * jnp.ones((1, 16), f32)

    t1, s1, d1, mxs, mxd = _kernel_a(xp, hmp, w1x, w1m, a1s, a1d)
    m16_1 = _m16(mxs, mxd, 8)
    o0, o1, dn0, dn1 = _gat_edges_64(src, dst, s1, d1, t1, m16_1, z64, z16)
    t2, s2, d2, mxs2, mxd2 = _kernel_b1(
        o0, o1, dn0, dn1, t1, s1, d1, m16_1[None], b1r, W2, a2s, a2d)

    m16_2 = _m16(mxs2, mxd2, 8)
    ar = jnp.arange(16)
    m16_2lo = jnp.where(ar < 4, m16_2, BIG)
    m16_2hi = jnp.where((ar >= 4) & (ar < 8), m16_2, BIG)
    olo0, olo1, dlo0, dlo1 = _gat_edges_l2lo(
        src, dst, s2, d2, t2[:, :64], m16_2lo, z64, z16)
    ohi0, ohi1, dhi0, dhi1 = _gat_edges_l2hi(
        src, dst, s2, d2, t2[:, 64:], m16_2hi, z64, z16)
    t3, s3, d3, mxs3, mxd3 = _kernel_b2(
        olo0, olo1, ohi0, ohi1, dlo0, dlo1, dhi0, dhi1,
        t2, s2, d2, m16_2[None], b2r, hop, gwh, gwp, w3p, a3s, a3d)

    m16_3 = _m16(mxs3, mxd3, 1)
    o0, o1, dn0, dn1 = _gat_edges_l3(src, dst, s3, d3, t3[:, :48],
                                     m16_3, z48, z16)
    o = _kernel_b3(o0, o1, dn0, dn1, t3, s3, d3, m16_3[None], b3r)
    return o[:N]
